# baseline (speedup 1.0000x reference)
import functools

import jax
import jax.numpy as jnp
from jax import lax
from jax.experimental import pallas as pl
from jax.experimental.pallas import tpu as pltpu
from jax.experimental.pallas import tpu_sc as plsc

VOCAB = 1000001
ROWS, COLS, EMBED = 16384, 26, 32

BLK_V = 512
N_BLK = (VOCAB + BLK_V - 1) // BLK_V
V_PAD = N_BLK * BLK_V
P_ROWS = V_PAD // 4


def _pack_body(t_ref, o_ref):
    x = t_ref[...]
    y = x.T.reshape(BLK_V // 4, 4, EMBED)
    o_ref[...] = jnp.concatenate([y[:, q, :] for q in range(4)], axis=1)


_pack = pl.pallas_call(
    _pack_body,
    grid=(N_BLK,),
    in_specs=[pl.BlockSpec((EMBED, BLK_V), lambda i: (0, i))],
    out_specs=pl.BlockSpec((BLK_V // 4, 128), lambda i: (i, 0)),
    out_shape=jax.ShapeDtypeStruct((P_ROWS, 128), jnp.float32),
)

B = ROWS * COLS
NW = 32
B_PER_W = B // NW
IDX_MINOR = 128
CH_J = 13
CHUNK = CH_J * IDX_MINOR
N_CHUNK = B_PER_W // CHUNK
ROWS_PER_W = B_PER_W // IDX_MINOR

_mesh = plsc.VectorSubcoreMesh(core_axis_name="c", subcore_axis_name="s")


@functools.partial(
    pl.kernel,
    mesh=_mesh,
    out_type=jax.ShapeDtypeStruct((B, EMBED), jnp.float32),
    scratch_types=[
        pltpu.VMEM((2, CH_J, IDX_MINOR), jnp.int32),
        pltpu.VMEM((2, CHUNK, EMBED), jnp.float32),
        pltpu.SemaphoreType.DMA,
        pltpu.SemaphoreType.DMA,
        pltpu.SemaphoreType.DMA,
    ],
    compiler_params=pltpu.CompilerParams(use_tc_tiling_on_sc=False),
)
def _embed_lookup(idx_hbm, table_hbm, out_hbm, idx_v, rows_v, gsem, osem0,
                  osem1):
    wid = lax.axis_index("s") * 2 + lax.axis_index("c")
    row_base = wid * ROWS_PER_W
    osems = (osem0, osem1)

    def do_chunk(c, b, wait_prev_store):
        row_off = row_base + c * CH_J
        flat_off = row_off * IDX_MINOR
        my_idx = idx_v.at[b]
        my_rows = rows_v.at[b]
        pltpu.sync_copy(idx_hbm.at[pl.ds(row_off, CH_J)], my_idx)
        if wait_prev_store:
            pltpu.make_async_copy(
                my_rows, out_hbm.at[pl.ds(flat_off, CHUNK)], osems[b]
            ).wait()
        copies = [
            pltpu.async_copy(
                table_hbm.at[my_idx.at[j]],
                my_rows.at[pl.ds(j * IDX_MINOR, IDX_MINOR)],
                gsem,
            )
            for j in range(CH_J)
        ]
        for cp in copies:
            cp.wait()
        pltpu.async_copy(my_rows, out_hbm.at[pl.ds(flat_off, CHUNK)],
                         osems[b])

    do_chunk(0, 0, False)
    do_chunk(1, 1, False)

    def pair_body(p, _):
        do_chunk(2 * p, 0, True)
        do_chunk(2 * p + 1, 1, True)
        return 0

    lax.fori_loop(1, N_CHUNK // 2, pair_body, 0)

    for b in range(2):
        pltpu.make_async_copy(
            rows_v.at[b], out_hbm.at[pl.ds(0, CHUNK)], osems[b]
        ).wait()


def kernel(ms, table):
    packed = _pack(table.T)
    table_lin = packed.reshape(V_PAD, EMBED)
    idx2d = ms.reshape(B // IDX_MINOR, IDX_MINOR)
    out = _embed_lookup(idx2d, table_lin)
    return out.reshape(ROWS, COLS, EMBED)


# speedup vs baseline: 1.7262x; 1.7262x over previous
import functools

import jax
import jax.numpy as jnp
from jax import lax
from jax.experimental import pallas as pl
from jax.experimental.pallas import tpu as pltpu
from jax.experimental.pallas import tpu_sc as plsc

ROWS, COLS, EMBED = 16384, 26, 32
B = ROWS * COLS
NW = 32
IDX_MINOR = 128
N_BT = ROWS // 128
BT_PER_W = N_BT // NW
CHUNK = 128 * COLS

_mesh = plsc.VectorSubcoreMesh(core_axis_name="c", subcore_axis_name="s")


@functools.partial(
    pl.kernel,
    mesh=_mesh,
    out_type=jax.ShapeDtypeStruct((COLS, 4, N_BT, 8, 128), jnp.float32),
    scratch_types=[
        pltpu.VMEM((COLS, IDX_MINOR), jnp.int32),
        pltpu.VMEM((CHUNK, EMBED), jnp.float32),
        pltpu.VMEM((4, 8, 128), jnp.float32),
        pltpu.SemaphoreType.DMA,
    ],
    compiler_params=pltpu.CompilerParams(use_tc_tiling_on_sc=False,
                                         needs_layout_passes=False),
)
def _embed_lookup(idx_hbm, table_hbm, out_hbm, idxb, rows_v, obuf, gsem):
    wid = lax.axis_index("s") * 2 + lax.axis_index("c")
    iot = lax.iota(jnp.int32, 16)
    r26 = iot * COLS

    def bt_body(bt, _):
        btg = wid * BT_PER_W + bt
        pltpu.sync_copy(idx_hbm.at[pl.ds(btg * COLS, COLS)], idxb)
        copies = [
            pltpu.async_copy(
                table_hbm.at[idxb.at[j]],
                rows_v.at[pl.ds(j * IDX_MINOR, IDX_MINOR)],
                gsem,
            )
            for j in range(COLS)
        ]
        for cp in copies:
            cp.wait()

        def c_body(c, _):
            ridx = [r26 + (c + 16 * COLS * k) for k in range(8)]
            for et in range(4):
                for es in range(8):
                    col = jnp.zeros((16,), jnp.int32) + (8 * et + es)
                    for k in range(8):
                        obuf[et, es, pl.ds(16 * k, 16)] = plsc.load_gather(
                            rows_v, [ridx[k], col])
            pltpu.sync_copy(obuf, out_hbm.at[c, :, btg, :, :])
            return 0

        lax.fori_loop(0, COLS, c_body, 0)
        return 0

    lax.fori_loop(0, BT_PER_W, bt_body, 0)


def kernel(ms, table):
    idx2d = ms.reshape(B // IDX_MINOR, IDX_MINOR)
    out5 = _embed_lookup(idx2d, table)
    return out5.transpose(2, 4, 0, 1, 3).reshape(ROWS, COLS, EMBED)


# speedup vs baseline: 1.9001x; 1.1007x over previous
import functools

import jax
import jax.numpy as jnp
from jax import lax
from jax.experimental import pallas as pl
from jax.experimental.pallas import tpu as pltpu
from jax.experimental.pallas import tpu_sc as plsc

ROWS, COLS, EMBED = 16384, 26, 32
B = ROWS * COLS
NW = 32
B_PER_W = B // NW
IDX_MINOR = 128
CH_J = 13
CHUNK = CH_J * IDX_MINOR
N_CHUNK = B_PER_W // CHUNK
ROWS_PER_W = B_PER_W // IDX_MINOR

_mesh = plsc.VectorSubcoreMesh(core_axis_name="c", subcore_axis_name="s")


@functools.partial(
    pl.kernel,
    mesh=_mesh,
    out_type=jax.ShapeDtypeStruct((B, EMBED), jnp.float32),
    scratch_types=[
        pltpu.VMEM((2, CH_J, IDX_MINOR), jnp.int32),
        pltpu.VMEM((2, CHUNK, EMBED), jnp.float32),
        pltpu.SemaphoreType.DMA,
        pltpu.SemaphoreType.DMA,
        pltpu.SemaphoreType.DMA,
    ],
    compiler_params=pltpu.CompilerParams(use_tc_tiling_on_sc=False),
)
def _embed_lookup(idx_hbm, table_hbm, out_hbm, idx_v, rows_v, gsem, osem0,
                  osem1):
    wid = lax.axis_index("s") * 2 + lax.axis_index("c")
    row_base = wid * ROWS_PER_W
    osems = (osem0, osem1)

    def do_chunk(c, b, wait_prev_store):
        row_off = row_base + c * CH_J
        flat_off = row_off * IDX_MINOR
        my_idx = idx_v.at[b]
        my_rows = rows_v.at[b]
        pltpu.sync_copy(idx_hbm.at[pl.ds(row_off, CH_J)], my_idx)
        if wait_prev_store:
            pltpu.make_async_copy(
                my_rows, out_hbm.at[pl.ds(flat_off, CHUNK)], osems[b]
            ).wait()
        copies = [
            pltpu.async_copy(
                table_hbm.at[my_idx.at[j]],
                my_rows.at[pl.ds(j * IDX_MINOR, IDX_MINOR)],
                gsem,
            )
            for j in range(CH_J)
        ]
        for cp in copies:
            cp.wait()
        pltpu.async_copy(my_rows, out_hbm.at[pl.ds(flat_off, CHUNK)],
                         osems[b])

    do_chunk(0, 0, False)
    do_chunk(1, 1, False)

    def pair_body(p, _):
        do_chunk(2 * p, 0, True)
        do_chunk(2 * p + 1, 1, True)
        return 0

    lax.fori_loop(1, N_CHUNK // 2, pair_body, 0)

    for b in range(2):
        pltpu.make_async_copy(
            rows_v.at[b], out_hbm.at[pl.ds(0, CHUNK)], osems[b]
        ).wait()


def kernel(ms, table):
    idx2d = ms.reshape(B // IDX_MINOR, IDX_MINOR)
    out = _embed_lookup(idx2d, table)
    return out.reshape(ROWS, COLS, EMBED)
